# Initial kernel scaffold; baseline (speedup 1.0000x reference)
import jax
import jax.numpy as jnp
from jax.experimental import pallas as pl


def kernel(indices, params):
    raise NotImplementedError("write your pallas kernel here")


# speedup vs baseline: 1.3519x; 1.3519x over previous
import jax
import jax.numpy as jnp
from jax import lax
from jax.experimental import pallas as pl
from jax.experimental.pallas import tpu as pltpu
from jax.experimental.pallas import tpu_sc as plsc

_DIM = 32
_BATCH = 16384
_FIELDS = 26

_B = _BATCH * _FIELDS
_NW = 32
_B_PER_W = _B // _NW
_CHUNK = 1024
_NCHUNK = _B_PER_W // _CHUNK


def _body(table_hbm, idx_hbm, out_hbm, idx_v, rows_v, gsem, ssem):
  nc = 2
  wid = lax.axis_index("s") * nc + lax.axis_index("c")
  base = wid * _B_PER_W
  pltpu.sync_copy(idx_hbm.at[pl.ds(base, _B_PER_W)], idx_v)

  def step(j, _):
    pltpu.async_copy(
        table_hbm.at[idx_v.at[pl.ds(j * _CHUNK, _CHUNK)]], rows_v, gsem
    ).wait()
    pltpu.async_copy(
        rows_v, out_hbm.at[pl.ds(base + j * _CHUNK, _CHUNK)], ssem
    ).wait()
    return ()

  lax.fori_loop(0, _NCHUNK, step, (), unroll=False)


@jax.jit
def kernel(indices, params):
  idx_flat = indices.reshape(_B).astype(jnp.int32)
  mesh = plsc.VectorSubcoreMesh(core_axis_name="c", subcore_axis_name="s")
  out = pl.kernel(
      _body,
      out_type=jax.ShapeDtypeStruct((_B, _DIM), jnp.float32),
      mesh=mesh,
      compiler_params=pltpu.CompilerParams(use_tc_tiling_on_sc=False),
      scratch_types=[
          pltpu.VMEM((_B_PER_W,), jnp.int32),
          pltpu.VMEM((_CHUNK, _DIM), jnp.float32),
          pltpu.SemaphoreType.DMA,
          pltpu.SemaphoreType.DMA,
      ],
  )(params, idx_flat)
  return out.reshape(_BATCH, _FIELDS, _DIM)


# speedup vs baseline: 1.3596x; 1.0058x over previous
import jax
import jax.numpy as jnp
from jax import lax
from jax.experimental import pallas as pl
from jax.experimental.pallas import tpu as pltpu
from jax.experimental.pallas import tpu_sc as plsc

_DIM = 32
_BATCH = 16384
_FIELDS = 26

_B = _BATCH * _FIELDS
_NW = 32
_B_PER_W = _B // _NW
_CHUNK = 1024
_NCHUNK = _B_PER_W // _CHUNK


def _body(table_hbm, idx_hbm, out_hbm, idx_v, rows0, rows1, gsem0, gsem1,
          ssem0, ssem1):
  nc = 2
  wid = lax.axis_index("s") * nc + lax.axis_index("c")
  base = wid * _B_PER_W
  rows = (rows0, rows1)
  gsem = (gsem0, gsem1)
  ssem = (ssem0, ssem1)
  pltpu.sync_copy(idx_hbm.at[pl.ds(base, _B_PER_W)], idx_v)

  def start_gather(j):
    return pltpu.async_copy(
        table_hbm.at[idx_v.at[pl.ds(j * _CHUNK, _CHUNK)]],
        rows[j % 2], gsem[j % 2])

  def start_store(j):
    return pltpu.async_copy(
        rows[j % 2], out_hbm.at[pl.ds(base + j * _CHUNK, _CHUNK)],
        ssem[j % 2])

  g = [None] * _NCHUNK
  s = [None] * _NCHUNK
  g[0] = start_gather(0)
  if _NCHUNK > 1:
    g[1] = start_gather(1)
  for j in range(_NCHUNK):
    g[j].wait()
    s[j] = start_store(j)
    if j + 2 < _NCHUNK:
      s[j].wait()
      g[j + 2] = start_gather(j + 2)
  for j in range(max(0, _NCHUNK - 2), _NCHUNK):
    s[j].wait()


@jax.jit
def kernel(indices, params):
  idx_flat = indices.reshape(_B).astype(jnp.int32)
  mesh = plsc.VectorSubcoreMesh(core_axis_name="c", subcore_axis_name="s")
  out = pl.kernel(
      _body,
      out_type=jax.ShapeDtypeStruct((_B, _DIM), jnp.float32),
      mesh=mesh,
      compiler_params=pltpu.CompilerParams(use_tc_tiling_on_sc=False),
      scratch_types=[
          pltpu.VMEM((_B_PER_W,), jnp.int32),
          pltpu.VMEM((_CHUNK, _DIM), jnp.float32),
          pltpu.VMEM((_CHUNK, _DIM), jnp.float32),
          pltpu.SemaphoreType.DMA,
          pltpu.SemaphoreType.DMA,
          pltpu.SemaphoreType.DMA,
          pltpu.SemaphoreType.DMA,
      ],
  )(params, idx_flat)
  return out.reshape(_BATCH, _FIELDS, _DIM)


# speedup vs baseline: 1.3606x; 1.0007x over previous
import functools

import jax
import jax.numpy as jnp
from jax import lax
from jax.experimental import pallas as pl
from jax.experimental.layout import Format, Layout
from jax.experimental.layout import with_layout_constraint
from jax.experimental.pallas import tpu as pltpu
from jax.experimental.pallas import tpu_sc as plsc

_NUM_SAMPLES = 1000000
_DIM = 32
_BATCH = 16384
_FIELDS = 26

_B = _BATCH * _FIELDS
_NW = 32
_B_PER_W = _B // _NW
_CHUNK = 1024
_NCHUNK = _B_PER_W // _CHUNK


def _body(table_hbm, idx_hbm, out_hbm, idx_v, rows0, rows1, gsem0, gsem1,
          ssem0, ssem1):
  nc = 2
  wid = lax.axis_index("s") * nc + lax.axis_index("c")
  base = wid * _B_PER_W
  rows = (rows0, rows1)
  gsem = (gsem0, gsem1)
  ssem = (ssem0, ssem1)
  pltpu.sync_copy(idx_hbm.at[pl.ds(base, _B_PER_W)], idx_v)

  def start_gather(j):
    return pltpu.async_copy(
        table_hbm.at[idx_v.at[pl.ds(j * _CHUNK, _CHUNK)]],
        rows[j % 2], gsem[j % 2])

  def start_store(j):
    return pltpu.async_copy(
        rows[j % 2], out_hbm.at[pl.ds(base + j * _CHUNK, _CHUNK)],
        ssem[j % 2])

  g = [None] * _NCHUNK
  s = [None] * _NCHUNK
  g[0] = start_gather(0)
  if _NCHUNK > 1:
    g[1] = start_gather(1)
  for j in range(_NCHUNK):
    g[j].wait()
    s[j] = start_store(j)
    if j + 2 < _NCHUNK:
      s[j].wait()
      g[j + 2] = start_gather(j + 2)
  for j in range(max(0, _NCHUNK - 2), _NCHUNK):
    s[j].wait()


@functools.lru_cache(maxsize=None)
def _jitted(sharding):
  del sharding
  row_major = Layout(major_to_minor=(0, 1), tiling=())

  @jax.jit
  def run(indices, params):
    idx_flat = indices.reshape(_B).astype(jnp.int32)
    tbl128 = lax.optimization_barrier(
        params.reshape(_NUM_SAMPLES * _DIM // 128, 128))
    tbl_flat = tbl128.reshape(_NUM_SAMPLES, _DIM)
    out = _pallas_gather(tbl_flat, idx_flat)
    return out.reshape(_BATCH, _FIELDS, _DIM)

  return run


def kernel(indices, params):
  sharding = getattr(params, "sharding", None)
  if sharding is None:
    sharding = jax.sharding.SingleDeviceSharding(jax.devices()[0])
  return _jitted(sharding)(indices, params)


def _pallas_gather(tbl_flat, idx_flat):
  mesh = plsc.VectorSubcoreMesh(core_axis_name="c", subcore_axis_name="s")
  out = pl.kernel(
      _body,
      out_type=jax.ShapeDtypeStruct((_B, _DIM), jnp.float32),
      mesh=mesh,
      compiler_params=pltpu.CompilerParams(use_tc_tiling_on_sc=False),
      scratch_types=[
          pltpu.VMEM((_B_PER_W,), jnp.int32),
          pltpu.VMEM((_CHUNK, _DIM), jnp.float32),
          pltpu.VMEM((_CHUNK, _DIM), jnp.float32),
          pltpu.SemaphoreType.DMA,
          pltpu.SemaphoreType.DMA,
          pltpu.SemaphoreType.DMA,
          pltpu.SemaphoreType.DMA,
      ],
  )(tbl_flat, idx_flat)
  return out
